# baseline (speedup 1.0000x reference)
import functools

import jax
import jax.numpy as jnp
from jax import lax
from jax.experimental import pallas as pl
from jax.experimental.pallas import tpu as pltpu
from jax.experimental.pallas import tpu_sc as plsc

ACC_C = 5
NCORES = 2
NSUB = 16
NTILES = NCORES * NSUB
LANES = 16
CHUNK_E = 1024


def _sc_segment_sums(ei, eat, xs0f, xs1f, B, N, NP, E):
    tiles_per_sample = NTILES // B
    full_chunks = E // CHUNK_E
    n_chunks = full_chunks // tiles_per_sample
    assert n_chunks * tiles_per_sample == full_chunks
    tail = E - full_chunks * CHUNK_E
    tail_off = full_chunks * CHUNK_E
    steps = CHUNK_E // LANES
    tail_steps = tail // LANES
    assert tail_steps * LANES == tail

    mesh = plsc.VectorSubcoreMesh(
        core_axis_name="c", subcore_axis_name="s",
        num_cores=NCORES, num_subcores=NSUB)

    @functools.partial(
        pl.kernel,
        out_type=jax.ShapeDtypeStruct((NTILES, ACC_C * NP), jnp.float32),
        mesh=mesh,
        compiler_params=pltpu.CompilerParams(
            needs_layout_passes=False, use_tc_tiling_on_sc=True),
        scratch_types=[
            pltpu.VMEM((ACC_C * NP,), jnp.float32),
            pltpu.VMEM((N,), jnp.float32),
            pltpu.VMEM((N,), jnp.float32),
            [pltpu.VMEM((2, CHUNK_E), jnp.int32) for _ in range(2)],
            [pltpu.VMEM((2, CHUNK_E), jnp.float32) for _ in range(2)],
            pltpu.VMEM((2, max(tail, LANES)), jnp.int32),
            pltpu.VMEM((2, max(tail, LANES)), jnp.float32),
            [pltpu.SemaphoreType.DMA for _ in range(6)],
        ],
    )
    def sc_kernel(ei_hbm, ea_hbm, x0_hbm, x1_hbm, out_hbm,
                  acc, xs0, xs1, eib, eab, tib, tab, sems):
        wid = lax.axis_index("s") * NCORES + lax.axis_index("c")
        b = wid // tiles_per_sample
        chunk = wid % tiles_per_sample
        base_chunk = chunk * n_chunks

        def start_chunk(ci, k):
            eoff = (base_chunk + ci) * CHUNK_E
            return [
                pltpu.async_copy(
                    ei_hbm.at[b, :, pl.ds(eoff, CHUNK_E)], eib[k], sems[k]),
                pltpu.async_copy(
                    ea_hbm.at[b, :, pl.ds(eoff, CHUNK_E)], eab[k], sems[2 + k]),
            ]

        hx0 = pltpu.async_copy(x0_hbm.at[pl.ds(b * N, N)], xs0, sems[4])
        hx1 = pltpu.async_copy(x1_hbm.at[pl.ds(b * N, N)], xs1, sems[5])
        pending = start_chunk(0, 0)

        zf = jnp.zeros((LANES,), jnp.float32)

        def zero_body(i, carry):
            for u in range(4):
                acc[pl.ds((i * 4 + u) * LANES, LANES)] = zf
            return carry

        lax.fori_loop(0, (ACC_C * NP) // (4 * LANES), zero_body, 0)

        hx0.wait()
        hx1.wait()

        ones_f = jnp.ones((LANES,), jnp.float32)

        def make_edge_body(eb, ab, unroll):
            def edge_body(i, carry):
                groups = []
                for u in range(unroll):
                    off = (i * unroll + u) * LANES
                    src16 = eb[0, pl.ds(off, LANES)]
                    dst16 = eb[1, pl.ds(off, LANES)]
                    ea0 = ab[0, pl.ds(off, LANES)]
                    ea1 = ab[1, pl.ds(off, LANES)]
                    xj0 = plsc.load_gather(xs0, [src16])
                    xj1 = plsc.load_gather(xs1, [src16])
                    groups.append((dst16, xj0, xj1, ea0, ea1))
                for dst16, xj0, xj1, ea0, ea1 in groups:
                    plsc.addupdate_scatter(acc, [dst16], ones_f)
                    plsc.addupdate_scatter(acc, [dst16 + NP], xj0)
                    plsc.addupdate_scatter(acc, [dst16 + 2 * NP], xj1)
                    plsc.addupdate_scatter(acc, [dst16 + 3 * NP], ea0)
                    plsc.addupdate_scatter(acc, [dst16 + 4 * NP], ea1)
                return carry
            return edge_body

        UNROLL = 4
        bodies = [make_edge_body(eib[k], eab[k], UNROLL) for k in range(2)]

        for ci in range(n_chunks):
            k = ci % 2
            for h in pending:
                h.wait()
            if ci + 1 < n_chunks:
                pending = start_chunk(ci + 1, 1 - k)
            lax.fori_loop(0, steps // UNROLL, bodies[k], 0)

        if tail:
            @pl.when(chunk == 0)
            def _tail():
                pltpu.sync_copy(ei_hbm.at[b, :, pl.ds(tail_off, tail)], tib)
                pltpu.sync_copy(ea_hbm.at[b, :, pl.ds(tail_off, tail)], tab)
                lax.fori_loop(0, tail_steps, make_edge_body(tib, tab, 1), 0)

        pltpu.sync_copy(acc, out_hbm.at[wid])

    return sc_kernel(ei, eat, xs0f, xs1f)


def _tc_combine(partials, loct, W5, Wd, B, N, NP, TPS):
    D = W5.shape[1]

    def body(p_ref, x_ref, w5_ref, wd_ref, o_ref):
        cols = []
        for c in range(ACC_C):
            cols.append(jnp.sum(p_ref[:, c * NP:(c + 1) * NP],
                                axis=0, keepdims=True))
        s = jnp.concatenate(cols, axis=0)
        r = 1.0 / jnp.maximum(s[0:1, :], 1.0)
        feat = s * r
        o1 = lax.dot_general(feat, w5_ref[...], (((0,), (0,)), ((), ())),
                             preferred_element_type=jnp.float32)
        xd = x_ref[0] * feat[0:1, 0:N]
        o2 = lax.dot_general(xd, wd_ref[...], (((0,), (0,)), ((), ())),
                             preferred_element_type=jnp.float32)
        o_ref[0] = o1[0:N] + o2

    return pl.pallas_call(
        body,
        grid=(B,),
        in_specs=[
            pl.BlockSpec((TPS, ACC_C * NP), lambda b: (b, 0)),
            pl.BlockSpec((1, 2, N), lambda b: (b, 0, 0)),
            pl.BlockSpec((ACC_C, D), lambda b: (0, 0)),
            pl.BlockSpec((2, D), lambda b: (0, 0)),
        ],
        out_specs=pl.BlockSpec((1, N, D), lambda b: (b, 0, 0)),
        out_shape=jax.ShapeDtypeStruct((B, N, D), jnp.float32),
    )(partials, loct, W5, Wd)


def kernel(locs, edge_index, edge_attr, W, b):
    B, N, _ = locs.shape
    E = edge_index.shape[2]
    TPS = NTILES // B
    NP = ((N + 127) // 128) * 128

    ei = edge_index.astype(jnp.int32)
    eat = jnp.transpose(edge_attr, (0, 2, 1))
    loct = jnp.transpose(locs, (0, 2, 1))
    xs0f = loct[:, 0, :].reshape(-1)
    xs1f = loct[:, 1, :].reshape(-1)

    partials = _sc_segment_sums(ei, eat, xs0f, xs1f, B, N, NP, E)

    W5 = jnp.concatenate([b[None, :], W[2:4], W[4:6]], axis=0)
    Wd = W[0:2]
    return _tc_combine(partials, loct, W5, Wd, B, N, NP, TPS)


# speedup vs baseline: 1.0494x; 1.0494x over previous
import functools

import jax
import jax.numpy as jnp
from jax import lax
from jax.experimental import pallas as pl
from jax.experimental.pallas import tpu as pltpu
from jax.experimental.pallas import tpu_sc as plsc

ACC_C = 5
NCORES = 2
NSUB = 16
NTILES = NCORES * NSUB
LANES = 16
CHUNK_E = 1024


def _sc_segment_sums(ei, eat, xs0f, xs1f, B, N, NP, E):
    tiles_per_sample = NTILES // B
    full_chunks = E // CHUNK_E
    n_chunks = full_chunks // tiles_per_sample
    assert n_chunks * tiles_per_sample == full_chunks
    tail = E - full_chunks * CHUNK_E
    tail_off = full_chunks * CHUNK_E
    steps = CHUNK_E // LANES
    tail_steps = tail // LANES
    assert tail_steps * LANES == tail

    mesh = plsc.VectorSubcoreMesh(
        core_axis_name="c", subcore_axis_name="s",
        num_cores=NCORES, num_subcores=NSUB)

    @functools.partial(
        pl.kernel,
        out_type=jax.ShapeDtypeStruct((NTILES, ACC_C * NP), jnp.float32),
        mesh=mesh,
        compiler_params=pltpu.CompilerParams(
            needs_layout_passes=False, use_tc_tiling_on_sc=True),
        scratch_types=[
            pltpu.VMEM((ACC_C * NP,), jnp.float32),
            pltpu.VMEM((N,), jnp.float32),
            pltpu.VMEM((N,), jnp.float32),
            [pltpu.VMEM((2, CHUNK_E), jnp.int32) for _ in range(2)],
            [pltpu.VMEM((2, CHUNK_E), jnp.float32) for _ in range(2)],
            pltpu.VMEM((2, max(tail, LANES)), jnp.int32),
            pltpu.VMEM((2, max(tail, LANES)), jnp.float32),
            [pltpu.SemaphoreType.DMA for _ in range(6)],
        ],
    )
    def sc_kernel(ei_hbm, ea_hbm, x0_hbm, x1_hbm, out_hbm,
                  acc, xs0, xs1, eib, eab, tib, tab, sems):
        wid = lax.axis_index("s") * NCORES + lax.axis_index("c")
        b = wid // tiles_per_sample
        chunk = wid % tiles_per_sample
        base_chunk = chunk * n_chunks

        def start_chunk(ci, k):
            eoff = (base_chunk + ci) * CHUNK_E
            return [
                pltpu.async_copy(
                    ei_hbm.at[b, :, pl.ds(eoff, CHUNK_E)], eib[k], sems[k]),
                pltpu.async_copy(
                    ea_hbm.at[b, :, pl.ds(eoff, CHUNK_E)], eab[k], sems[2 + k]),
            ]

        hx0 = pltpu.async_copy(x0_hbm.at[pl.ds(b * N, N)], xs0, sems[4])
        hx1 = pltpu.async_copy(x1_hbm.at[pl.ds(b * N, N)], xs1, sems[5])
        pending = start_chunk(0, 0)

        zf = jnp.zeros((LANES,), jnp.float32)

        def zero_body(i, carry):
            for u in range(4):
                acc[pl.ds((i * 4 + u) * LANES, LANES)] = zf
            return carry

        lax.fori_loop(0, (ACC_C * NP) // (4 * LANES), zero_body, 0)

        hx0.wait()
        hx1.wait()

        ones_f = jnp.ones((LANES,), jnp.float32)

        def make_edge_body(eb, ab, unroll):
            def edge_body(i, carry):
                groups = []
                for u in range(unroll):
                    off = (i * unroll + u) * LANES
                    src16 = eb[0, pl.ds(off, LANES)]
                    dst16 = eb[1, pl.ds(off, LANES)]
                    ea0 = ab[0, pl.ds(off, LANES)]
                    ea1 = ab[1, pl.ds(off, LANES)]
                    xj0 = plsc.load_gather(xs0, [src16])
                    xj1 = plsc.load_gather(xs1, [src16])
                    groups.append((dst16, xj0, xj1, ea0, ea1))
                for dst16, xj0, xj1, ea0, ea1 in groups:
                    plsc.addupdate_scatter(acc, [dst16], ones_f)
                    plsc.addupdate_scatter(acc, [dst16 + NP], xj0)
                    plsc.addupdate_scatter(acc, [dst16 + 2 * NP], xj1)
                    plsc.addupdate_scatter(acc, [dst16 + 3 * NP], ea0)
                    plsc.addupdate_scatter(acc, [dst16 + 4 * NP], ea1)
                return carry
            return edge_body

        UNROLL = 2
        bodies = [make_edge_body(eib[k], eab[k], UNROLL) for k in range(2)]

        for ci in range(n_chunks):
            k = ci % 2
            for h in pending:
                h.wait()
            if ci + 1 < n_chunks:
                pending = start_chunk(ci + 1, 1 - k)
            lax.fori_loop(0, steps // UNROLL, bodies[k], 0)

        if tail:
            @pl.when(chunk == 0)
            def _tail():
                pltpu.sync_copy(ei_hbm.at[b, :, pl.ds(tail_off, tail)], tib)
                pltpu.sync_copy(ea_hbm.at[b, :, pl.ds(tail_off, tail)], tab)
                lax.fori_loop(0, tail_steps, make_edge_body(tib, tab, 1), 0)

        pltpu.sync_copy(acc, out_hbm.at[wid])

    return sc_kernel(ei, eat, xs0f, xs1f)


def _tc_combine(partials, loct, W5, Wd, B, N, NP, TPS):
    D = W5.shape[1]

    def body(p_ref, x_ref, w5_ref, wd_ref, o_ref):
        cols = []
        for c in range(ACC_C):
            cols.append(jnp.sum(p_ref[:, c * NP:(c + 1) * NP],
                                axis=0, keepdims=True))
        s = jnp.concatenate(cols, axis=0)
        r = 1.0 / jnp.maximum(s[0:1, :], 1.0)
        feat = s * r
        o1 = lax.dot_general(feat, w5_ref[...], (((0,), (0,)), ((), ())),
                             preferred_element_type=jnp.float32)
        xd = x_ref[0] * feat[0:1, 0:N]
        o2 = lax.dot_general(xd, wd_ref[...], (((0,), (0,)), ((), ())),
                             preferred_element_type=jnp.float32)
        o_ref[0] = o1[0:N] + o2

    return pl.pallas_call(
        body,
        grid=(B,),
        in_specs=[
            pl.BlockSpec((TPS, ACC_C * NP), lambda b: (b, 0)),
            pl.BlockSpec((1, 2, N), lambda b: (b, 0, 0)),
            pl.BlockSpec((ACC_C, D), lambda b: (0, 0)),
            pl.BlockSpec((2, D), lambda b: (0, 0)),
        ],
        out_specs=pl.BlockSpec((1, N, D), lambda b: (b, 0, 0)),
        out_shape=jax.ShapeDtypeStruct((B, N, D), jnp.float32),
    )(partials, loct, W5, Wd)


def kernel(locs, edge_index, edge_attr, W, b):
    B, N, _ = locs.shape
    E = edge_index.shape[2]
    TPS = NTILES // B
    NP = ((N + 127) // 128) * 128

    ei = edge_index.astype(jnp.int32)
    eat = jnp.transpose(edge_attr, (0, 2, 1))
    loct = jnp.transpose(locs, (0, 2, 1))
    xs0f = loct[:, 0, :].reshape(-1)
    xs1f = loct[:, 1, :].reshape(-1)

    partials = _sc_segment_sums(ei, eat, xs0f, xs1f, B, N, NP, E)

    W5 = jnp.concatenate([b[None, :], W[2:4], W[4:6]], axis=0)
    Wd = W[0:2]
    return _tc_combine(partials, loct, W5, Wd, B, N, NP, TPS)
